# baseline (speedup 1.0000x reference)
import jax
import jax.numpy as jnp
from jax import lax
from jax.experimental import pallas as pl
from jax.experimental.pallas import tpu as pltpu
from jax.experimental.pallas import tpu_sc as plsc

N = 10000
E = 160000
H = 256
NC, NS, L = 2, 16, 16
NW = NC * NS
CHUNK = 128
NCHUNK = E // CHUNK
KMAX = (NCHUNK + NW - 1) // NW
N4 = N * 4
HW = H // 2
BE = 4000
NORM_INV = 1.0 / 100.0
CRANGE = 100.0


def _proj_body(h_ref, w_ref, out_ref):
    out_ref[0] = jnp.dot(
        h_ref[...], w_ref[0], preferred_element_type=jnp.float32
    ).astype(jnp.bfloat16)


def _project(h, w_stack):
    return pl.pallas_call(
        _proj_body,
        grid=(2,),
        in_specs=[
            pl.BlockSpec((N, H), lambda j: (0, 0)),
            pl.BlockSpec((1, H, H), lambda j: (j, 0, 0)),
        ],
        out_specs=pl.BlockSpec((1, N, H), lambda j: (j, 0, 0)),
        out_shape=jax.ShapeDtypeStruct((2, N, H), jnp.bfloat16),
    )(h, w_stack)


def _gather_body(t_hbm, rcp_hbm, g_hbm,
                 rcall_v, buf0, buf1, semg0, semg1, sems0, sems1):
    wid = lax.axis_index("s") * NC + lax.axis_index("c")
    pltpu.sync_copy(rcp_hbm.at[wid], rcall_v)

    def g_start(p, buf, sem):
        ca = pltpu.async_copy(t_hbm.at[rcall_v.at[p * 2]],
                              buf.at[pl.ds(0, CHUNK), :], sem)
        cb = pltpu.async_copy(t_hbm.at[rcall_v.at[p * 2 + 1]],
                              buf.at[pl.ds(CHUNK, CHUNK), :], sem)
        return ca, cb

    def g_add(buf):
        def add_body(e, c2):
            for j in range(HW // L):
                s = pl.ds(j * L, L)
                a = plsc.bitcast(buf[e, s], jnp.bfloat16)
                b = plsc.bitcast(buf[e + CHUNK, s], jnp.bfloat16)
                buf[e, s] = plsc.bitcast(a + b, jnp.int32)
            return c2

        lax.fori_loop(0, CHUNK, add_body, 0)

    def g_store(p, buf, sem):
        base = pl.multiple_of((p * NW + wid) * CHUNK, CHUNK)
        return pltpu.async_copy(buf.at[pl.ds(0, CHUNK), :],
                                g_hbm.at[pl.ds(base, CHUNK), :], sem)

    def do_pair(k2, carry):
        p0 = k2 * 2
        ca0, cb0 = g_start(p0, buf0, semg0)
        ca1, cb1 = g_start(p0 + 1, buf1, semg1)
        ca0.wait()
        cb0.wait()
        g_add(buf0)
        st0 = g_store(p0, buf0, sems0)
        ca1.wait()
        cb1.wait()
        g_add(buf1)
        st1 = g_store(p0 + 1, buf1, sems1)
        st0.wait()
        st1.wait()
        return carry

    lax.fori_loop(0, (KMAX - 2) // 2, do_pair, 0)

    p38 = KMAX - 2
    ca0, cb0 = g_start(p38, buf0, semg0)
    ca0.wait()
    cb0.wait()
    g_add(buf0)
    g_store(p38, buf0, sems0).wait()

    @pl.when((KMAX - 1) * NW + wid < NCHUNK)
    def _():
        ca1, cb1 = g_start(KMAX - 1, buf1, semg1)
        ca1.wait()
        cb1.wait()
        g_add(buf1)
        g_store(KMAX - 1, buf1, sems1).wait()


def _gather(t_table, rcp):
    mesh = plsc.VectorSubcoreMesh(
        core_axis_name="c", subcore_axis_name="s",
        num_cores=NC, num_subcores=NS)
    f = pl.kernel(
        _gather_body,
        out_type=jax.ShapeDtypeStruct((E, HW), jnp.int32),
        mesh=mesh,
        scratch_types=[
            pltpu.VMEM((KMAX * 2, CHUNK), jnp.int32),
            pltpu.VMEM((2 * CHUNK, HW), jnp.int32),
            pltpu.VMEM((2 * CHUNK, HW), jnp.int32),
            pltpu.SemaphoreType.DMA,
            pltpu.SemaphoreType.DMA,
            pltpu.SemaphoreType.DMA,
            pltpu.SemaphoreType.DMA,
        ],
        compiler_params=pltpu.CompilerParams(needs_layout_passes=False),
    )
    return f(t_table, rcp)


def _mlp_body(g_ref, ea_ref, cd_ref, w1c_ref, b1_ref, w2_ref, b2_ref, w3_ref,
              out_ref):
    z = (g_ref[...].astype(jnp.float32)
         + jnp.dot(ea_ref[...], w1c_ref[...], preferred_element_type=jnp.float32)
         + b1_ref[...])
    x1 = z * jax.nn.sigmoid(z)
    y = jnp.dot(x1.astype(jnp.bfloat16), w2_ref[...],
                preferred_element_type=jnp.float32) + b2_ref[...]
    x2 = y * jax.nn.sigmoid(y)
    m = jnp.sum(x2 * w3_ref[...], axis=1, keepdims=True)
    out_ref[...] = cd_ref[...] * (jnp.tanh(m) * CRANGE)


def _mlp(g, ea8, cd4, w1c8, b1r, w2t, b2r, w3r):
    grid = (E // BE,)
    return pl.pallas_call(
        _mlp_body,
        grid=grid,
        in_specs=[
            pl.BlockSpec((BE, H), lambda i: (i, 0)),
            pl.BlockSpec((BE, 8), lambda i: (i, 0)),
            pl.BlockSpec((BE, 4), lambda i: (i, 0)),
            pl.BlockSpec((8, H), lambda i: (0, 0)),
            pl.BlockSpec((1, H), lambda i: (0, 0)),
            pl.BlockSpec((H, H), lambda i: (0, 0)),
            pl.BlockSpec((1, H), lambda i: (0, 0)),
            pl.BlockSpec((1, H), lambda i: (0, 0)),
        ],
        out_specs=pl.BlockSpec((BE, 4), lambda i: (i, 0)),
        out_shape=jax.ShapeDtypeStruct((E, 4), jnp.float32),
    )(g, ea8, cd4, w1c8, b1r, w2t, b2r, w3r)


ZSEG = 1000
EPT = E // NW
NSUB = EPT // CHUNK
TAIL = EPT - NSUB * CHUNK


def _scatter_body(t_hbm, sidx_hbm, sidxt_hbm, zeros_hbm, part_hbm,
                  sidx_v, sidxt_v, tv0, tv1, tvt, shared, sem):
    cid = lax.axis_index("c")
    sid = lax.axis_index("s")
    wid = sid * NC + cid
    ebase = pl.multiple_of(wid * EPT, 8)

    @pl.when(sid < 10)
    def _():
        pltpu.sync_copy(zeros_hbm.at[pl.ds(sid * ZSEG, ZSEG), :],
                        shared.at[pl.ds(sid * ZSEG, ZSEG), :])

    pltpu.sync_copy(sidx_hbm.at[wid], sidx_v)
    pltpu.sync_copy(sidxt_hbm.at[wid], sidxt_v)
    plsc.subcore_barrier()

    def sub_body(k2, carry):
        k0 = k2 * 2
        pltpu.sync_copy(t_hbm.at[pl.ds(ebase + k0 * CHUNK, CHUNK), :], tv0)
        cp0 = pltpu.async_copy(tv0, shared.at[sidx_v.at[k0]], sem, add=True)
        pltpu.sync_copy(t_hbm.at[pl.ds(ebase + (k0 + 1) * CHUNK, CHUNK), :],
                        tv1)
        cp1 = pltpu.async_copy(tv1, shared.at[sidx_v.at[k0 + 1]], sem,
                               add=True)
        cp0.wait()
        cp1.wait()
        return carry

    lax.fori_loop(0, (NSUB - 1) // 2, sub_body, 0)
    pltpu.sync_copy(t_hbm.at[pl.ds(ebase + (NSUB - 1) * CHUNK, CHUNK), :], tv0)
    cp0 = pltpu.async_copy(tv0, shared.at[sidx_v.at[NSUB - 1]], sem, add=True)
    pltpu.sync_copy(t_hbm.at[pl.ds(ebase + NSUB * CHUNK, TAIL), :], tvt)
    cp1 = pltpu.async_copy(tvt, shared.at[sidxt_v], sem, add=True)
    cp0.wait()
    cp1.wait()

    plsc.subcore_barrier()

    @pl.when(sid < 10)
    def _():
        pltpu.sync_copy(shared.at[pl.ds(sid * ZSEG, ZSEG), :],
                        part_hbm.at[cid, pl.ds(sid * ZSEG, ZSEG), :])


def _scatter(t4, sidx, sidxt, zeros2d):
    mesh = plsc.VectorSubcoreMesh(
        core_axis_name="c", subcore_axis_name="s",
        num_cores=NC, num_subcores=NS)
    f = pl.kernel(
        _scatter_body,
        out_type=jax.ShapeDtypeStruct((NC, N, 4), jnp.float32),
        mesh=mesh,
        scratch_types=[
            pltpu.VMEM((NSUB, CHUNK), jnp.int32),
            pltpu.VMEM((TAIL,), jnp.int32),
            pltpu.VMEM((CHUNK, 4), jnp.float32),
            pltpu.VMEM((CHUNK, 4), jnp.float32),
            pltpu.VMEM((TAIL, 4), jnp.float32),
            pltpu.VMEM_SHARED((N, 4), jnp.float32),
            pltpu.SemaphoreType.DMA,
        ],
        compiler_params=pltpu.CompilerParams(needs_layout_passes=False),
    )
    return f(t4, sidx, sidxt, zeros2d)


def _final_body(p_ref, c_ref, out_ref):
    s = jnp.sum(p_ref[...], axis=0, keepdims=True)
    out_ref[...] = c_ref[...] + s * NORM_INV


def _final(part, coordp):
    return pl.pallas_call(
        _final_body,
        in_specs=[
            pl.BlockSpec((NC, N4), lambda: (0, 0)),
            pl.BlockSpec((1, N4), lambda: (0, 0)),
        ],
        out_specs=pl.BlockSpec((1, N4), lambda: (0, 0)),
        out_shape=jax.ShapeDtypeStruct((1, N4), jnp.float32),
    )(part, coordp)


def kernel(h, coord, edge_index, coord_diff, edge_attr, w1, b1, w2, b2, w3):
    row = edge_index[0].astype(jnp.int32)
    col = edge_index[1].astype(jnp.int32)

    w_stack = jnp.stack([w1[:, :H].T, w1[:, H:2 * H].T])
    t_bf = _project(h, w_stack).reshape(2 * N, HW, 2)
    t_i32 = lax.bitcast_convert_type(t_bf, jnp.int32)

    rc = jnp.stack([row.reshape(NCHUNK, CHUNK),
                    (col + N).reshape(NCHUNK, CHUNK)], axis=1)
    rcp = jnp.pad(rc, ((0, NW * KMAX - NCHUNK), (0, 0), (0, 0)))
    rcp = rcp.reshape(KMAX, NW, 2, CHUNK).transpose(1, 0, 2, 3)
    rcp = rcp.reshape(NW, KMAX * 2, CHUNK)
    g_i32 = _gather(t_i32, rcp)
    g = lax.bitcast_convert_type(g_i32, jnp.bfloat16).reshape(E, H)

    w1c8 = jnp.pad(w1[:, 2 * H:].T, ((0, 4), (0, 0)))
    ea8 = jnp.pad(edge_attr, ((0, 0), (0, 4)))
    cd4 = jnp.pad(coord_diff, ((0, 0), (0, 1)))
    t4 = _mlp(g, ea8, cd4, w1c8,
              b1.reshape(1, H), w2.T.astype(jnp.bfloat16),
              b2.reshape(1, H), w3)

    zeros2d = jnp.zeros((N, 4), jnp.float32)
    rpt = row.reshape(NW, EPT)
    sidx = rpt[:, :NSUB * CHUNK].reshape(NW, NSUB, CHUNK)
    sidxt = rpt[:, NSUB * CHUNK:]
    part = _scatter(t4, sidx, sidxt, zeros2d).reshape(NC, N4)

    coordp = jnp.pad(coord, ((0, 0), (0, 1))).reshape(1, N4)
    out = _final(part, coordp)
    return out.reshape(-1, 4)[:N, :3]


# speedup vs baseline: 1.6132x; 1.6132x over previous
import jax
import jax.numpy as jnp
from jax import lax
from jax.experimental import pallas as pl
from jax.experimental.pallas import tpu as pltpu
from jax.experimental.pallas import tpu_sc as plsc

N = 10000
E = 160000
H = 256
NC, NS, L = 2, 16, 16
NW = NC * NS
CHUNK = 128
NCHUNK = E // CHUNK
KMAX = (NCHUNK + NW - 1) // NW
N4 = N * 4
HW = H // 2
BE = 4000
NORM_INV = 1.0 / 100.0
CRANGE = 100.0


def _proj_body(h_ref, we_ref, wo_ref, out_ref):
    ye = jnp.dot(h_ref[...], we_ref[0], preferred_element_type=jnp.float32)
    yo = jnp.dot(h_ref[...], wo_ref[0], preferred_element_type=jnp.float32)
    eb = lax.bitcast_convert_type(
        ye.astype(jnp.bfloat16).astype(jnp.float32), jnp.uint32)
    ob = lax.bitcast_convert_type(
        yo.astype(jnp.bfloat16).astype(jnp.float32), jnp.uint32)
    packed = (lax.shift_right_logical(eb, jnp.uint32(16))
              | (ob & jnp.uint32(0xFFFF0000)))
    out_ref[0] = lax.bitcast_convert_type(packed, jnp.int32)


def _project(h, w_even, w_odd):
    return pl.pallas_call(
        _proj_body,
        grid=(2,),
        in_specs=[
            pl.BlockSpec((N, H), lambda j: (0, 0)),
            pl.BlockSpec((1, H, HW), lambda j: (j, 0, 0)),
            pl.BlockSpec((1, H, HW), lambda j: (j, 0, 0)),
        ],
        out_specs=pl.BlockSpec((1, N, HW), lambda j: (j, 0, 0)),
        out_shape=jax.ShapeDtypeStruct((2, N, HW), jnp.int32),
    )(h, w_even, w_odd)


def _gather_body(t_hbm, rcp_hbm, g_hbm,
                 rcall_v, buf0, buf1, semg0, semg1, sems0, sems1):
    wid = lax.axis_index("s") * NC + lax.axis_index("c")
    pltpu.sync_copy(rcp_hbm.at[wid], rcall_v)

    def g_start(p, buf, sem):
        ca = pltpu.async_copy(t_hbm.at[rcall_v.at[p * 2]],
                              buf.at[pl.ds(0, CHUNK), :], sem)
        cb = pltpu.async_copy(t_hbm.at[rcall_v.at[p * 2 + 1]],
                              buf.at[pl.ds(CHUNK, CHUNK), :], sem)
        return ca, cb

    def g_add(buf):
        def add_body(e, c2):
            for j in range(HW // L):
                s = pl.ds(j * L, L)
                a = plsc.bitcast(buf[e, s], jnp.bfloat16)
                b = plsc.bitcast(buf[e + CHUNK, s], jnp.bfloat16)
                buf[e, s] = plsc.bitcast(a + b, jnp.int32)
            return c2

        lax.fori_loop(0, CHUNK, add_body, 0)

    def g_store(p, buf, sem):
        chunk = wid * KMAX + p
        base = pl.multiple_of(
            jnp.where(chunk < NCHUNK, chunk, NCHUNK) * CHUNK, CHUNK)
        return pltpu.async_copy(buf.at[pl.ds(0, CHUNK), :],
                                g_hbm.at[pl.ds(base, CHUNK), :], sem)

    def do_pair(k2, carry):
        p0 = k2 * 2
        ca0, cb0 = g_start(p0, buf0, semg0)
        ca1, cb1 = g_start(p0 + 1, buf1, semg1)
        ca0.wait()
        cb0.wait()
        g_add(buf0)
        st0 = g_store(p0, buf0, sems0)
        ca1.wait()
        cb1.wait()
        g_add(buf1)
        st1 = g_store(p0 + 1, buf1, sems1)
        st0.wait()
        st1.wait()
        return carry

    lax.fori_loop(0, KMAX // 2, do_pair, 0)


def _gather(t_table, rcp):
    mesh = plsc.VectorSubcoreMesh(
        core_axis_name="c", subcore_axis_name="s",
        num_cores=NC, num_subcores=NS)
    f = pl.kernel(
        _gather_body,
        out_type=jax.ShapeDtypeStruct((E + CHUNK, HW), jnp.int32),
        mesh=mesh,
        scratch_types=[
            pltpu.VMEM((KMAX * 2, CHUNK), jnp.int32),
            pltpu.VMEM((2 * CHUNK, HW), jnp.int32),
            pltpu.VMEM((2 * CHUNK, HW), jnp.int32),
            pltpu.SemaphoreType.DMA,
            pltpu.SemaphoreType.DMA,
            pltpu.SemaphoreType.DMA,
            pltpu.SemaphoreType.DMA,
        ],
        compiler_params=pltpu.CompilerParams(needs_layout_passes=False),
    )
    return f(t_table, rcp)


def _mlp_body(g_ref, ea_ref, cd_ref, w1ce_ref, w1co_ref, b1e_ref, b1o_ref,
              w2e_ref, w2o_ref, b2_ref, w3_ref, out_ref):
    gw = lax.bitcast_convert_type(g_ref[...], jnp.uint32)
    ge = lax.bitcast_convert_type(lax.shift_left(gw, jnp.uint32(16)),
                                  jnp.float32)
    go = lax.bitcast_convert_type(gw & jnp.uint32(0xFFFF0000), jnp.float32)
    ea = ea_ref[...]
    ze = (ge + jnp.dot(ea, w1ce_ref[...], preferred_element_type=jnp.float32)
          + b1e_ref[...])
    zo = (go + jnp.dot(ea, w1co_ref[...], preferred_element_type=jnp.float32)
          + b1o_ref[...])
    x1e = ze * jax.nn.sigmoid(ze)
    x1o = zo * jax.nn.sigmoid(zo)
    y = (jnp.dot(x1e.astype(jnp.bfloat16), w2e_ref[...],
                 preferred_element_type=jnp.float32)
         + jnp.dot(x1o.astype(jnp.bfloat16), w2o_ref[...],
                   preferred_element_type=jnp.float32)
         + b2_ref[...])
    x2 = y * jax.nn.sigmoid(y)
    m = jnp.sum(x2 * w3_ref[...], axis=1, keepdims=True)
    out_ref[...] = cd_ref[...] * (jnp.tanh(m) * CRANGE)


def _mlp(g, ea8, cd4, w1ce, w1co, b1e, b1o, w2e, w2o, b2r, w3r):
    grid = (E // BE,)
    return pl.pallas_call(
        _mlp_body,
        grid=grid,
        in_specs=[
            pl.BlockSpec((BE, HW), lambda i: (i, 0)),
            pl.BlockSpec((BE, 8), lambda i: (i, 0)),
            pl.BlockSpec((BE, 4), lambda i: (i, 0)),
            pl.BlockSpec((8, HW), lambda i: (0, 0)),
            pl.BlockSpec((8, HW), lambda i: (0, 0)),
            pl.BlockSpec((1, HW), lambda i: (0, 0)),
            pl.BlockSpec((1, HW), lambda i: (0, 0)),
            pl.BlockSpec((HW, H), lambda i: (0, 0)),
            pl.BlockSpec((HW, H), lambda i: (0, 0)),
            pl.BlockSpec((1, H), lambda i: (0, 0)),
            pl.BlockSpec((1, H), lambda i: (0, 0)),
        ],
        out_specs=pl.BlockSpec((BE, 4), lambda i: (i, 0)),
        out_shape=jax.ShapeDtypeStruct((E, 4), jnp.float32),
    )(g, ea8, cd4, w1ce, w1co, b1e, b1o, w2e, w2o, b2r, w3r)


ZSEG = 1000
EPT = E // NW
NSUB = EPT // CHUNK
TAIL = EPT - NSUB * CHUNK


def _scatter_body(t_hbm, sidx_hbm, sidxt_hbm, zeros_hbm, part_hbm,
                  sidx_v, sidxt_v, tv0, tv1, tvt, shared, sem):
    cid = lax.axis_index("c")
    sid = lax.axis_index("s")
    wid = sid * NC + cid
    ebase = pl.multiple_of(wid * EPT, 8)

    @pl.when(sid < 10)
    def _():
        pltpu.sync_copy(zeros_hbm.at[pl.ds(sid * ZSEG, ZSEG), :],
                        shared.at[pl.ds(sid * ZSEG, ZSEG), :])

    pltpu.sync_copy(sidx_hbm.at[wid], sidx_v)
    pltpu.sync_copy(sidxt_hbm.at[wid], sidxt_v)
    plsc.subcore_barrier()

    def sub_body(k2, carry):
        k0 = k2 * 2
        pltpu.sync_copy(t_hbm.at[pl.ds(ebase + k0 * CHUNK, CHUNK), :], tv0)
        cp0 = pltpu.async_copy(tv0, shared.at[sidx_v.at[k0]], sem, add=True)
        pltpu.sync_copy(t_hbm.at[pl.ds(ebase + (k0 + 1) * CHUNK, CHUNK), :],
                        tv1)
        cp1 = pltpu.async_copy(tv1, shared.at[sidx_v.at[k0 + 1]], sem,
                               add=True)
        cp0.wait()
        cp1.wait()
        return carry

    lax.fori_loop(0, (NSUB - 1) // 2, sub_body, 0)
    pltpu.sync_copy(t_hbm.at[pl.ds(ebase + (NSUB - 1) * CHUNK, CHUNK), :], tv0)
    cp0 = pltpu.async_copy(tv0, shared.at[sidx_v.at[NSUB - 1]], sem, add=True)
    pltpu.sync_copy(t_hbm.at[pl.ds(ebase + NSUB * CHUNK, TAIL), :], tvt)
    cp1 = pltpu.async_copy(tvt, shared.at[sidxt_v], sem, add=True)
    cp0.wait()
    cp1.wait()

    plsc.subcore_barrier()

    @pl.when(sid < 10)
    def _():
        pltpu.sync_copy(shared.at[pl.ds(sid * ZSEG, ZSEG), :],
                        part_hbm.at[cid, pl.ds(sid * ZSEG, ZSEG), :])


def _scatter(t4, sidx, sidxt, zeros2d):
    mesh = plsc.VectorSubcoreMesh(
        core_axis_name="c", subcore_axis_name="s",
        num_cores=NC, num_subcores=NS)
    f = pl.kernel(
        _scatter_body,
        out_type=jax.ShapeDtypeStruct((NC, N, 4), jnp.float32),
        mesh=mesh,
        scratch_types=[
            pltpu.VMEM((NSUB, CHUNK), jnp.int32),
            pltpu.VMEM((TAIL,), jnp.int32),
            pltpu.VMEM((CHUNK, 4), jnp.float32),
            pltpu.VMEM((CHUNK, 4), jnp.float32),
            pltpu.VMEM((TAIL, 4), jnp.float32),
            pltpu.VMEM_SHARED((N, 4), jnp.float32),
            pltpu.SemaphoreType.DMA,
        ],
        compiler_params=pltpu.CompilerParams(needs_layout_passes=False),
    )
    return f(t4, sidx, sidxt, zeros2d)


def _final_body(p_ref, c_ref, out_ref):
    s = jnp.sum(p_ref[...], axis=0, keepdims=True)
    out_ref[...] = c_ref[...] + s * NORM_INV


def _final(part, coordp):
    return pl.pallas_call(
        _final_body,
        in_specs=[
            pl.BlockSpec((NC, N4), lambda: (0, 0)),
            pl.BlockSpec((1, N4), lambda: (0, 0)),
        ],
        out_specs=pl.BlockSpec((1, N4), lambda: (0, 0)),
        out_shape=jax.ShapeDtypeStruct((1, N4), jnp.float32),
    )(part, coordp)


def kernel(h, coord, edge_index, coord_diff, edge_attr, w1, b1, w2, b2, w3):
    row = edge_index[0].astype(jnp.int32)
    col = edge_index[1].astype(jnp.int32)

    w1at = w1[:, :H].T
    w1bt = w1[:, H:2 * H].T
    w_even = jnp.stack([w1at[:, 0::2], w1bt[:, 0::2]])
    w_odd = jnp.stack([w1at[:, 1::2], w1bt[:, 1::2]])
    t_i32 = _project(h, w_even, w_odd).reshape(2 * N, HW)

    rc = jnp.stack([row.reshape(NCHUNK, CHUNK),
                    (col + N).reshape(NCHUNK, CHUNK)], axis=1)
    rcp = jnp.pad(rc, ((0, NW * KMAX - NCHUNK), (0, 0), (0, 0)))
    rcp = rcp.reshape(NW, KMAX * 2, CHUNK)
    g_i32 = _gather(t_i32, rcp)

    w1c8 = jnp.pad(w1[:, 2 * H:].T, ((0, 4), (0, 0)))
    ea8 = jnp.pad(edge_attr, ((0, 0), (0, 4)))
    cd4 = jnp.pad(coord_diff, ((0, 0), (0, 1)))
    t4 = _mlp(g_i32, ea8, cd4,
              w1c8[:, 0::2], w1c8[:, 1::2],
              b1[0::2].reshape(1, HW), b1[1::2].reshape(1, HW),
              w2.T[0::2, :].astype(jnp.bfloat16),
              w2.T[1::2, :].astype(jnp.bfloat16),
              b2.reshape(1, H), w3)

    zeros2d = jnp.zeros((N, 4), jnp.float32)
    rpt = row.reshape(NW, EPT)
    sidx = rpt[:, :NSUB * CHUNK].reshape(NW, NSUB, CHUNK)
    sidxt = rpt[:, NSUB * CHUNK:]
    part = _scatter(t4, sidx, sidxt, zeros2d).reshape(NC, N4)

    coordp = jnp.pad(coord, ((0, 0), (0, 1))).reshape(1, N4)
    out = _final(part, coordp)
    return out.reshape(-1, 4)[:N, :3]


# speedup vs baseline: 2.5564x; 1.5847x over previous
import jax
import jax.numpy as jnp
from jax import lax
from jax.experimental import pallas as pl
from jax.experimental.pallas import tpu as pltpu
from jax.experimental.pallas import tpu_sc as plsc

N = 10000
E = 160000
H = 256
NC, NS, L = 2, 16, 16
NW = NC * NS
CHUNK = 128
NCHUNK = E // CHUNK
KMAX = (NCHUNK + NW - 1) // NW
N4 = N * 4
HW = H // 2
BE = 4000
NORM_INV = 1.0 / 100.0
CRANGE = 100.0


def _proj_body(h_ref, we_ref, wo_ref, out_ref):
    ye = jnp.dot(h_ref[...], we_ref[0], preferred_element_type=jnp.float32)
    yo = jnp.dot(h_ref[...], wo_ref[0], preferred_element_type=jnp.float32)
    eb = lax.bitcast_convert_type(
        ye.astype(jnp.bfloat16).astype(jnp.float32), jnp.uint32)
    ob = lax.bitcast_convert_type(
        yo.astype(jnp.bfloat16).astype(jnp.float32), jnp.uint32)
    packed = (lax.shift_right_logical(eb, jnp.uint32(16))
              | (ob & jnp.uint32(0xFFFF0000)))
    out_ref[0] = lax.bitcast_convert_type(packed, jnp.int32)


def _project(h, w_even, w_odd):
    return pl.pallas_call(
        _proj_body,
        grid=(2,),
        in_specs=[
            pl.BlockSpec((N, H), lambda j: (0, 0)),
            pl.BlockSpec((1, H, HW), lambda j: (j, 0, 0)),
            pl.BlockSpec((1, H, HW), lambda j: (j, 0, 0)),
        ],
        out_specs=pl.BlockSpec((1, N, HW), lambda j: (j, 0, 0)),
        out_shape=jax.ShapeDtypeStruct((2, N, HW), jnp.int32),
    )(h, w_even, w_odd)


def _gather_body(t_hbm, rcp_hbm, g_hbm,
                 rcall_v, buf0, buf1, semg0, semg1, sems0, sems1):
    wid = lax.axis_index("s") * NC + lax.axis_index("c")
    pltpu.sync_copy(rcp_hbm.at[wid], rcall_v)

    def g_start(p, buf, sem):
        ca = pltpu.async_copy(t_hbm.at[rcall_v.at[p * 2]],
                              buf.at[pl.ds(0, CHUNK), :], sem)
        cb = pltpu.async_copy(t_hbm.at[rcall_v.at[p * 2 + 1]],
                              buf.at[pl.ds(CHUNK, CHUNK), :], sem)
        return ca, cb

    def g_add(buf):
        def add_body(e, c2):
            for j in range(HW // L):
                s = pl.ds(j * L, L)
                a = plsc.bitcast(buf[e, s], jnp.bfloat16)
                b = plsc.bitcast(buf[e + CHUNK, s], jnp.bfloat16)
                buf[e, s] = plsc.bitcast(a + b, jnp.int32)
            return c2

        lax.fori_loop(0, CHUNK, add_body, 0)

    def g_store(p, buf, sem):
        chunk = wid * KMAX + p
        base = pl.multiple_of(
            jnp.where(chunk < NCHUNK, chunk, NCHUNK) * CHUNK, CHUNK)
        return pltpu.async_copy(buf.at[pl.ds(0, CHUNK), :],
                                g_hbm.at[pl.ds(base, CHUNK), :], sem)

    def do_pair(k2, carry):
        p0 = k2 * 2
        ca0, cb0 = g_start(p0, buf0, semg0)
        ca1, cb1 = g_start(p0 + 1, buf1, semg1)
        ca0.wait()
        cb0.wait()
        g_add(buf0)
        st0 = g_store(p0, buf0, sems0)
        ca1.wait()
        cb1.wait()
        g_add(buf1)
        st1 = g_store(p0 + 1, buf1, sems1)
        st0.wait()
        st1.wait()
        return carry

    lax.fori_loop(0, KMAX // 2, do_pair, 0)


def _gather(t_table, rcp):
    mesh = plsc.VectorSubcoreMesh(
        core_axis_name="c", subcore_axis_name="s",
        num_cores=NC, num_subcores=NS)
    f = pl.kernel(
        _gather_body,
        out_type=jax.ShapeDtypeStruct((E + CHUNK, HW), jnp.int32),
        mesh=mesh,
        scratch_types=[
            pltpu.VMEM((KMAX * 2, CHUNK), jnp.int32),
            pltpu.VMEM((2 * CHUNK, HW), jnp.int32),
            pltpu.VMEM((2 * CHUNK, HW), jnp.int32),
            pltpu.SemaphoreType.DMA,
            pltpu.SemaphoreType.DMA,
            pltpu.SemaphoreType.DMA,
            pltpu.SemaphoreType.DMA,
        ],
        compiler_params=pltpu.CompilerParams(needs_layout_passes=False),
    )
    return f(t_table, rcp)


def _mlp_body(g_ref, ea_ref, cd_ref, w1ce_ref, w1co_ref, b1e_ref, b1o_ref,
              w2e_ref, w2o_ref, b2_ref, w3_ref, out_ref):
    gw = lax.bitcast_convert_type(g_ref[...], jnp.uint32)
    ge = lax.bitcast_convert_type(lax.shift_left(gw, jnp.uint32(16)),
                                  jnp.float32)
    go = lax.bitcast_convert_type(gw & jnp.uint32(0xFFFF0000), jnp.float32)
    ea = ea_ref[...]
    ze = (ge + jnp.dot(ea, w1ce_ref[...], preferred_element_type=jnp.float32)
          + b1e_ref[...])
    zo = (go + jnp.dot(ea, w1co_ref[...], preferred_element_type=jnp.float32)
          + b1o_ref[...])
    x1e = ze * jax.nn.sigmoid(ze)
    x1o = zo * jax.nn.sigmoid(zo)
    y = (jnp.dot(x1e.astype(jnp.bfloat16), w2e_ref[...],
                 preferred_element_type=jnp.float32)
         + jnp.dot(x1o.astype(jnp.bfloat16), w2o_ref[...],
                   preferred_element_type=jnp.float32)
         + b2_ref[...])
    x2 = y * jax.nn.sigmoid(y)
    m = jnp.sum(x2 * w3_ref[...], axis=1, keepdims=True)
    out_ref[...] = cd_ref[...] * (jnp.tanh(m) * CRANGE)


def _mlp(g, ea8, cd4, w1ce, w1co, b1e, b1o, w2e, w2o, b2r, w3r):
    grid = (E // BE,)
    return pl.pallas_call(
        _mlp_body,
        grid=grid,
        in_specs=[
            pl.BlockSpec((BE, HW), lambda i: (i, 0)),
            pl.BlockSpec((BE, 8), lambda i: (i, 0)),
            pl.BlockSpec((BE, 4), lambda i: (i, 0)),
            pl.BlockSpec((8, HW), lambda i: (0, 0)),
            pl.BlockSpec((8, HW), lambda i: (0, 0)),
            pl.BlockSpec((1, HW), lambda i: (0, 0)),
            pl.BlockSpec((1, HW), lambda i: (0, 0)),
            pl.BlockSpec((HW, H), lambda i: (0, 0)),
            pl.BlockSpec((HW, H), lambda i: (0, 0)),
            pl.BlockSpec((1, H), lambda i: (0, 0)),
            pl.BlockSpec((1, H), lambda i: (0, 0)),
        ],
        out_specs=pl.BlockSpec((BE, 4), lambda i: (i, 0)),
        out_shape=jax.ShapeDtypeStruct((E, 4), jnp.float32),
    )(g, ea8, cd4, w1ce, w1co, b1e, b1o, w2e, w2o, b2r, w3r)


ZSEG = 1000
EPT = E // NW
NSUB = EPT // CHUNK
TAIL = EPT - NSUB * CHUNK


def _scatter_body(t_hbm, sidx_hbm, sidxt_hbm, zeros_hbm, part_hbm,
                  sidx_v, sidxt_v, tv0, tv1, tvt, shared, sem):
    cid = lax.axis_index("c")
    sid = lax.axis_index("s")
    wid = sid * NC + cid
    ebase = pl.multiple_of(wid * EPT, 8)

    @pl.when(sid < 10)
    def _():
        pltpu.sync_copy(zeros_hbm.at[pl.ds(sid * ZSEG, ZSEG), :],
                        shared.at[pl.ds(sid * ZSEG, ZSEG), :])

    pltpu.sync_copy(sidx_hbm.at[wid], sidx_v)
    pltpu.sync_copy(sidxt_hbm.at[wid], sidxt_v)
    plsc.subcore_barrier()

    def sub_body(k2, carry):
        k0 = k2 * 2
        pltpu.sync_copy(t_hbm.at[pl.ds(ebase + k0 * CHUNK, CHUNK), :], tv0)
        cp0 = pltpu.async_copy(tv0, shared.at[sidx_v.at[k0]], sem, add=True)
        pltpu.sync_copy(t_hbm.at[pl.ds(ebase + (k0 + 1) * CHUNK, CHUNK), :],
                        tv1)
        cp1 = pltpu.async_copy(tv1, shared.at[sidx_v.at[k0 + 1]], sem,
                               add=True)
        cp0.wait()
        cp1.wait()
        return carry

    lax.fori_loop(0, (NSUB - 1) // 2, sub_body, 0)
    pltpu.sync_copy(t_hbm.at[pl.ds(ebase + (NSUB - 1) * CHUNK, CHUNK), :], tv0)
    cp0 = pltpu.async_copy(tv0, shared.at[sidx_v.at[NSUB - 1]], sem, add=True)
    pltpu.sync_copy(t_hbm.at[pl.ds(ebase + NSUB * CHUNK, TAIL), :], tvt)
    cp1 = pltpu.async_copy(tvt, shared.at[sidxt_v], sem, add=True)
    cp0.wait()
    cp1.wait()

    plsc.subcore_barrier()

    @pl.when(sid < 10)
    def _():
        pltpu.sync_copy(shared.at[pl.ds(sid * ZSEG, ZSEG), :],
                        part_hbm.at[cid, pl.ds(sid * ZSEG, ZSEG), :])


def _scatter(t4, sidx, sidxt, zeros2d):
    mesh = plsc.VectorSubcoreMesh(
        core_axis_name="c", subcore_axis_name="s",
        num_cores=NC, num_subcores=NS)
    f = pl.kernel(
        _scatter_body,
        out_type=jax.ShapeDtypeStruct((NC, N, 4), jnp.float32),
        mesh=mesh,
        scratch_types=[
            pltpu.VMEM((NSUB, CHUNK), jnp.int32),
            pltpu.VMEM((TAIL,), jnp.int32),
            pltpu.VMEM((CHUNK, 4), jnp.float32),
            pltpu.VMEM((CHUNK, 4), jnp.float32),
            pltpu.VMEM((TAIL, 4), jnp.float32),
            pltpu.VMEM_SHARED((N, 4), jnp.float32),
            pltpu.SemaphoreType.DMA,
        ],
        compiler_params=pltpu.CompilerParams(needs_layout_passes=False),
    )
    return f(t4, sidx, sidxt, zeros2d)


def _final_body(p_ref, c_ref, out_ref):
    s = jnp.sum(p_ref[...], axis=0, keepdims=True)
    out_ref[...] = c_ref[...] + s * NORM_INV


def _final(part, coordp):
    return pl.pallas_call(
        _final_body,
        in_specs=[
            pl.BlockSpec((NC, N4), lambda: (0, 0)),
            pl.BlockSpec((1, N4), lambda: (0, 0)),
        ],
        out_specs=pl.BlockSpec((1, N4), lambda: (0, 0)),
        out_shape=jax.ShapeDtypeStruct((1, N4), jnp.float32),
    )(part, coordp)


def kernel(h, coord, edge_index, coord_diff, edge_attr, w1, b1, w2, b2, w3):
    row = edge_index[0].astype(jnp.int32)
    col = edge_index[1].astype(jnp.int32)

    w1at = w1[:, :H].T
    w1bt = w1[:, H:2 * H].T
    w_even = jnp.stack([w1at[:, 0::2], w1bt[:, 0::2]])
    w_odd = jnp.stack([w1at[:, 1::2], w1bt[:, 1::2]])
    t_i32 = _project(h, w_even, w_odd).reshape(2 * N, HW)

    rc = jnp.stack([row.reshape(NCHUNK, CHUNK),
                    (col + N).reshape(NCHUNK, CHUNK)], axis=1)
    npad = NW * KMAX - NCHUNK
    pad_idx = (jnp.arange(npad * 2 * CHUNK, dtype=jnp.int32)
               % (2 * N)).reshape(npad, 2, CHUNK)
    rcp = jnp.concatenate([rc, pad_idx], axis=0)
    rcp = rcp.reshape(NW, KMAX * 2, CHUNK)
    g_i32 = _gather(t_i32, rcp)

    w1c8 = jnp.pad(w1[:, 2 * H:].T, ((0, 4), (0, 0)))
    ea8 = jnp.pad(edge_attr, ((0, 0), (0, 4)))
    cd4 = jnp.pad(coord_diff, ((0, 0), (0, 1)))
    t4 = _mlp(g_i32, ea8, cd4,
              w1c8[:, 0::2], w1c8[:, 1::2],
              b1[0::2].reshape(1, HW), b1[1::2].reshape(1, HW),
              w2.T[0::2, :].astype(jnp.bfloat16),
              w2.T[1::2, :].astype(jnp.bfloat16),
              b2.reshape(1, H), w3)

    zeros2d = jnp.zeros((N, 4), jnp.float32)
    rpt = row.reshape(NW, EPT)
    sidx = rpt[:, :NSUB * CHUNK].reshape(NW, NSUB, CHUNK)
    sidxt = rpt[:, NSUB * CHUNK:]
    part = _scatter(t4, sidx, sidxt, zeros2d).reshape(NC, N4)

    coordp = jnp.pad(coord, ((0, 0), (0, 1))).reshape(1, N4)
    out = _final(part, coordp)
    return out.reshape(-1, 4)[:N, :3]


# speedup vs baseline: 3.1725x; 1.2410x over previous
import jax
import jax.numpy as jnp
from jax import lax
from jax.experimental import pallas as pl
from jax.experimental.pallas import tpu as pltpu
from jax.experimental.pallas import tpu_sc as plsc

N = 10000
E = 160000
H = 256
NC, NS, L = 2, 16, 16
NW = NC * NS
CHUNK = 128
NCHUNK = E // CHUNK
KMAX = (NCHUNK + NW - 1) // NW
N4 = N * 4
HW = H // 2
BE = 4000
NORM_INV = 1.0 / 100.0
CRANGE = 100.0


def _proj_body(h_ref, we_ref, wo_ref, out_ref):
    ye = jnp.dot(h_ref[...], we_ref[0], preferred_element_type=jnp.float32)
    yo = jnp.dot(h_ref[...], wo_ref[0], preferred_element_type=jnp.float32)
    eb = lax.bitcast_convert_type(
        ye.astype(jnp.bfloat16).astype(jnp.float32), jnp.uint32)
    ob = lax.bitcast_convert_type(
        yo.astype(jnp.bfloat16).astype(jnp.float32), jnp.uint32)
    packed = (lax.shift_right_logical(eb, jnp.uint32(16))
              | (ob & jnp.uint32(0xFFFF0000)))
    out_ref[0] = lax.bitcast_convert_type(packed, jnp.int32)


def _project(h, w_even, w_odd):
    return pl.pallas_call(
        _proj_body,
        grid=(2,),
        in_specs=[
            pl.BlockSpec((N, H), lambda j: (0, 0)),
            pl.BlockSpec((1, H, HW), lambda j: (j, 0, 0)),
            pl.BlockSpec((1, H, HW), lambda j: (j, 0, 0)),
        ],
        out_specs=pl.BlockSpec((1, N, HW), lambda j: (j, 0, 0)),
        out_shape=jax.ShapeDtypeStruct((2, N, HW), jnp.int32),
    )(h, w_even, w_odd)


def _gather_body(t_hbm, rcp_hbm, g_hbm,
                 rcall_v, buf0, buf1, semg0, semg1, sems0, sems1):
    wid = lax.axis_index("s") * NC + lax.axis_index("c")
    pltpu.sync_copy(rcp_hbm.at[wid], rcall_v)

    def g_start(p, buf, sem):
        ca = pltpu.async_copy(t_hbm.at[rcall_v.at[p * 2]],
                              buf.at[pl.ds(0, CHUNK), :], sem)
        cb = pltpu.async_copy(t_hbm.at[rcall_v.at[p * 2 + 1]],
                              buf.at[pl.ds(CHUNK, CHUNK), :], sem)
        return ca, cb

    def g_add(buf):
        def add_body(e, c2):
            for j in range(HW // L):
                s = pl.ds(j * L, L)
                a = plsc.bitcast(buf[e, s], jnp.bfloat16)
                b = plsc.bitcast(buf[e + CHUNK, s], jnp.bfloat16)
                buf[e, s] = plsc.bitcast(a + b, jnp.int32)
            return c2

        lax.fori_loop(0, CHUNK, add_body, 0)

    def g_store(p, buf, sem):
        chunk = wid * KMAX + p
        base = pl.multiple_of(
            jnp.where(chunk < NCHUNK, chunk, NCHUNK) * CHUNK, CHUNK)
        return pltpu.async_copy(buf.at[pl.ds(0, CHUNK), :],
                                g_hbm.at[pl.ds(base, CHUNK), :], sem)

    def do_pair(k2, carry):
        p0 = k2 * 2
        ca0, cb0 = g_start(p0, buf0, semg0)
        ca1, cb1 = g_start(p0 + 1, buf1, semg1)
        ca0.wait()
        cb0.wait()
        g_add(buf0)
        st0 = g_store(p0, buf0, sems0)
        ca1.wait()
        cb1.wait()
        g_add(buf1)
        st1 = g_store(p0 + 1, buf1, sems1)
        st0.wait()
        st1.wait()
        return carry

    lax.fori_loop(0, KMAX // 2, do_pair, 0)


def _gather(t_table, rcp):
    mesh = plsc.VectorSubcoreMesh(
        core_axis_name="c", subcore_axis_name="s",
        num_cores=NC, num_subcores=NS)
    f = pl.kernel(
        _gather_body,
        out_type=jax.ShapeDtypeStruct((E + BE, HW), jnp.int32),
        mesh=mesh,
        scratch_types=[
            pltpu.VMEM((KMAX * 2, CHUNK), jnp.int32),
            pltpu.VMEM((2 * CHUNK, HW), jnp.int32),
            pltpu.VMEM((2 * CHUNK, HW), jnp.int32),
            pltpu.SemaphoreType.DMA,
            pltpu.SemaphoreType.DMA,
            pltpu.SemaphoreType.DMA,
            pltpu.SemaphoreType.DMA,
        ],
        compiler_params=pltpu.CompilerParams(needs_layout_passes=False),
    )
    return f(t_table, rcp)


def _mlp_body(g_ref, ea_ref, cd_ref, w1ce_ref, w1co_ref, b1e_ref, b1o_ref,
              w2e_ref, w2o_ref, b2_ref, w3_ref, out_ref):
    gw = lax.bitcast_convert_type(g_ref[...], jnp.uint32)
    ge = lax.bitcast_convert_type(lax.shift_left(gw, jnp.uint32(16)),
                                  jnp.float32)
    go = lax.bitcast_convert_type(gw & jnp.uint32(0xFFFF0000), jnp.float32)
    ea = ea_ref[...]
    ze = (ge + jnp.dot(ea, w1ce_ref[...], preferred_element_type=jnp.float32)
          + b1e_ref[...])
    zo = (go + jnp.dot(ea, w1co_ref[...], preferred_element_type=jnp.float32)
          + b1o_ref[...])
    x1e = ze * jax.nn.sigmoid(ze)
    x1o = zo * jax.nn.sigmoid(zo)
    y = (jnp.dot(x1e.astype(jnp.bfloat16), w2e_ref[...],
                 preferred_element_type=jnp.float32)
         + jnp.dot(x1o.astype(jnp.bfloat16), w2o_ref[...],
                   preferred_element_type=jnp.float32)
         + b2_ref[...])
    x2 = y * jax.nn.sigmoid(y)
    m = jnp.sum(x2 * w3_ref[...], axis=1, keepdims=True)
    cd4 = jnp.pad(cd_ref[...], ((0, 0), (0, 1)))
    out_ref[...] = cd4 * (jnp.tanh(m) * CRANGE)


def _mlp(g, ea, cd, w1ce, w1co, b1e, b1o, w2e, w2o, b2r, w3r):
    grid = ((E + BE) // BE,)
    return pl.pallas_call(
        _mlp_body,
        grid=grid,
        in_specs=[
            pl.BlockSpec((BE, HW), lambda i: (i, 0)),
            pl.BlockSpec((BE, 4), lambda i: (i, 0)),
            pl.BlockSpec((BE, 3), lambda i: (i, 0)),
            pl.BlockSpec((4, HW), lambda i: (0, 0)),
            pl.BlockSpec((4, HW), lambda i: (0, 0)),
            pl.BlockSpec((1, HW), lambda i: (0, 0)),
            pl.BlockSpec((1, HW), lambda i: (0, 0)),
            pl.BlockSpec((HW, H), lambda i: (0, 0)),
            pl.BlockSpec((HW, H), lambda i: (0, 0)),
            pl.BlockSpec((1, H), lambda i: (0, 0)),
            pl.BlockSpec((1, H), lambda i: (0, 0)),
        ],
        out_specs=pl.BlockSpec((BE, 4), lambda i: (i, 0)),
        out_shape=jax.ShapeDtypeStruct((E + BE, 4), jnp.float32),
    )(g, ea, cd, w1ce, w1co, b1e, b1o, w2e, w2o, b2r, w3r)


ZSEG = 1000
EPT = E // NW
NSUB = EPT // CHUNK
TAIL = EPT - NSUB * CHUNK


def _scatter_body(t_hbm, sidx_hbm, sidxt_hbm, zeros_hbm, part_hbm,
                  sidx_v, sidxt_v, tv0, tv1, tvt, shared, sem):
    cid = lax.axis_index("c")
    sid = lax.axis_index("s")
    wid = sid * NC + cid
    ebase = pl.multiple_of(wid * EPT, 8)

    @pl.when(sid < 10)
    def _():
        pltpu.sync_copy(zeros_hbm.at[pl.ds(sid * ZSEG, ZSEG), :],
                        shared.at[pl.ds(sid * ZSEG, ZSEG), :])

    pltpu.sync_copy(sidx_hbm.at[wid], sidx_v)
    pltpu.sync_copy(sidxt_hbm.at[wid], sidxt_v)
    plsc.subcore_barrier()

    def sub_body(k2, carry):
        k0 = k2 * 2
        pltpu.sync_copy(t_hbm.at[pl.ds(ebase + k0 * CHUNK, CHUNK), :], tv0)
        cp0 = pltpu.async_copy(tv0, shared.at[sidx_v.at[k0]], sem, add=True)
        pltpu.sync_copy(t_hbm.at[pl.ds(ebase + (k0 + 1) * CHUNK, CHUNK), :],
                        tv1)
        cp1 = pltpu.async_copy(tv1, shared.at[sidx_v.at[k0 + 1]], sem,
                               add=True)
        cp0.wait()
        cp1.wait()
        return carry

    lax.fori_loop(0, (NSUB - 1) // 2, sub_body, 0)
    pltpu.sync_copy(t_hbm.at[pl.ds(ebase + (NSUB - 1) * CHUNK, CHUNK), :], tv0)
    cp0 = pltpu.async_copy(tv0, shared.at[sidx_v.at[NSUB - 1]], sem, add=True)
    pltpu.sync_copy(t_hbm.at[pl.ds(ebase + NSUB * CHUNK, TAIL), :], tvt)
    cp1 = pltpu.async_copy(tvt, shared.at[sidxt_v], sem, add=True)
    cp0.wait()
    cp1.wait()

    plsc.subcore_barrier()

    @pl.when(sid < 10)
    def _():
        pltpu.sync_copy(shared.at[pl.ds(sid * ZSEG, ZSEG), :],
                        part_hbm.at[cid, pl.ds(sid * ZSEG, ZSEG), :])


def _scatter(t4, sidx, sidxt, zeros2d):
    mesh = plsc.VectorSubcoreMesh(
        core_axis_name="c", subcore_axis_name="s",
        num_cores=NC, num_subcores=NS)
    f = pl.kernel(
        _scatter_body,
        out_type=jax.ShapeDtypeStruct((NC, N, 4), jnp.float32),
        mesh=mesh,
        scratch_types=[
            pltpu.VMEM((NSUB, CHUNK), jnp.int32),
            pltpu.VMEM((TAIL,), jnp.int32),
            pltpu.VMEM((CHUNK, 4), jnp.float32),
            pltpu.VMEM((CHUNK, 4), jnp.float32),
            pltpu.VMEM((TAIL, 4), jnp.float32),
            pltpu.VMEM_SHARED((N, 4), jnp.float32),
            pltpu.SemaphoreType.DMA,
        ],
        compiler_params=pltpu.CompilerParams(needs_layout_passes=False),
    )
    return f(t4, sidx, sidxt, zeros2d)


def _final_body(p_ref, c_ref, out_ref):
    s = jnp.sum(p_ref[...], axis=0, keepdims=True)
    out_ref[...] = c_ref[...] + s * NORM_INV


def _final(part, coordp):
    return pl.pallas_call(
        _final_body,
        in_specs=[
            pl.BlockSpec((NC, N4), lambda: (0, 0)),
            pl.BlockSpec((1, N4), lambda: (0, 0)),
        ],
        out_specs=pl.BlockSpec((1, N4), lambda: (0, 0)),
        out_shape=jax.ShapeDtypeStruct((1, N4), jnp.float32),
    )(part, coordp)


def kernel(h, coord, edge_index, coord_diff, edge_attr, w1, b1, w2, b2, w3):
    row = edge_index[0].astype(jnp.int32)
    col = edge_index[1].astype(jnp.int32)

    w1at = w1[:, :H].T
    w1bt = w1[:, H:2 * H].T
    w_even = jnp.stack([w1at[:, 0::2], w1bt[:, 0::2]])
    w_odd = jnp.stack([w1at[:, 1::2], w1bt[:, 1::2]])
    t_i32 = _project(h, w_even, w_odd).reshape(2 * N, HW)

    rc = jnp.stack([row.reshape(NCHUNK, CHUNK),
                    (col + N).reshape(NCHUNK, CHUNK)], axis=1)
    npad = NW * KMAX - NCHUNK
    pad_idx = (jnp.arange(npad * 2 * CHUNK, dtype=jnp.int32)
               % (2 * N)).reshape(npad, 2, CHUNK)
    rcp = jnp.concatenate([rc, pad_idx], axis=0)
    rcp = rcp.reshape(NW, KMAX * 2, CHUNK)
    g_i32 = _gather(t_i32, rcp)

    w1ct = w1[:, 2 * H:].T
    t4 = _mlp(g_i32, edge_attr, coord_diff,
              w1ct[:, 0::2], w1ct[:, 1::2],
              b1[0::2].reshape(1, HW), b1[1::2].reshape(1, HW),
              w2.T[0::2, :].astype(jnp.bfloat16),
              w2.T[1::2, :].astype(jnp.bfloat16),
              b2.reshape(1, H), w3)

    zeros2d = jnp.zeros((N, 4), jnp.float32)
    rpt = row.reshape(NW, EPT)
    sidx = rpt[:, :NSUB * CHUNK].reshape(NW, NSUB, CHUNK)
    sidxt = rpt[:, NSUB * CHUNK:]
    part = _scatter(t4, sidx, sidxt, zeros2d).reshape(NC, N4)

    coordp = jnp.pad(coord, ((0, 0), (0, 1))).reshape(1, N4)
    out = _final(part, coordp)
    return out.reshape(-1, 4)[:N, :3]
